# baseline (speedup 1.0000x reference)
import jax
import jax.numpy as jnp
from jax.experimental import pallas as pl
from jax.experimental.pallas import tpu as pltpu

L = 4096
BLK = 256


def _tc_body(tab_ref, idx_ref, out_ref):
    idx = idx_ref[...]
    t = [tab_ref[0, k] for k in range(16)]
    b0 = (idx & 1) == 1
    l0 = [jnp.where(b0, t[2 * k + 1], t[2 * k]) for k in range(8)]
    b1 = (idx & 2) == 2
    l1 = [jnp.where(b1, l0[2 * k + 1], l0[2 * k]) for k in range(4)]
    b2 = (idx & 4) == 4
    l2 = [jnp.where(b2, l1[2 * k + 1], l1[2 * k]) for k in range(2)]
    b3 = (idx & 8) == 8
    out_ref[...] = jnp.where(b3, l2[1], l2[0])


def _tc_lookup(table16, idx):
    return pl.pallas_call(
        _tc_body,
        grid=(L // BLK,),
        in_specs=[
            pl.BlockSpec((1, 16), lambda i: (0, 0)),
            pl.BlockSpec((BLK, L), lambda i: (i, 0)),
        ],
        out_specs=pl.BlockSpec((BLK, L), lambda i: (i, 0)),
        out_shape=jax.ShapeDtypeStruct((L, L), jnp.float32),
    )(table16.reshape(1, 16), idx)


def kernel(selected_ids, crf_transitions_model):
    idx = selected_ids.astype(jnp.int32)
    flat = crf_transitions_model.reshape(-1)
    table16 = jnp.concatenate([flat, jnp.zeros((1,), jnp.float32)])
    return _tc_lookup(table16, idx)
